# baseline (device time: 99480 ns/iter reference)
import jax
import jax.numpy as jnp
from jax import lax
from jax.experimental import pallas as pl
from jax.experimental.pallas import tpu as pltpu


def kernel(x, assign, W1, W2):
    t, d = x.shape
    n_loc, _, f = W1.shape

    x_bf = x.astype(jnp.bfloat16)
    w1_bf = W1.astype(jnp.bfloat16)
    w2_bf = W2.astype(jnp.bfloat16)
    a2 = assign.reshape(t, 1)

    def body(x_ref, a_ref, w1_ref, w2_ref, out_ref,
             xr_buf, ar_buf, rs_buf, rr_buf, send_sems, recv_sems):
        my_x = lax.axis_index("x")
        my_y = lax.axis_index("y")
        my_z = lax.axis_index("z")
        peer = (my_x, 1 - my_y, my_z)

        barrier_sem = pltpu.get_barrier_semaphore()
        pl.semaphore_signal(barrier_sem, inc=1, device_id=peer,
                            device_id_type=pl.DeviceIdType.MESH)
        pl.semaphore_wait(barrier_sem, 1)

        rdma_x = pltpu.make_async_remote_copy(
            src_ref=x_ref, dst_ref=xr_buf,
            send_sem=send_sems.at[0], recv_sem=recv_sems.at[0],
            device_id=peer, device_id_type=pl.DeviceIdType.MESH)
        rdma_a = pltpu.make_async_remote_copy(
            src_ref=a_ref, dst_ref=ar_buf,
            send_sem=send_sems.at[1], recv_sem=recv_sems.at[1],
            device_id=peer, device_id_type=pl.DeviceIdType.MESH)
        rdma_x.start()
        rdma_a.start()
        rdma_x.wait()
        rdma_a.wait()

        e0 = 2 * my_y

        def ffn(xm, e):
            h = jnp.maximum(
                jnp.dot(xm, w1_ref[e], preferred_element_type=jnp.float32),
                0.0).astype(jnp.bfloat16)
            return jnp.dot(h, w2_ref[e], preferred_element_type=jnp.float32)

        def moe(xs, a):
            acc = jnp.zeros((t, d), jnp.float32)
            for i in range(n_loc):
                acc = acc + ffn(jnp.where(a == e0 + i, xs, 0), i)
            return acc

        rs_buf[...] = moe(xr_buf[...], ar_buf[...]).astype(jnp.bfloat16)
        rdma_r = pltpu.make_async_remote_copy(
            src_ref=rs_buf, dst_ref=rr_buf,
            send_sem=send_sems.at[2], recv_sem=recv_sems.at[2],
            device_id=peer, device_id_type=pl.DeviceIdType.MESH)
        rdma_r.start()

        acc_local = moe(x_ref[...], a_ref[...])

        rdma_r.wait()
        out_ref[...] = acc_local + rr_buf[...].astype(jnp.float32)

    return pl.pallas_call(
        body,
        out_shape=jax.ShapeDtypeStruct((t, d), jnp.float32),
        in_specs=[pl.BlockSpec(memory_space=pltpu.VMEM)] * 4,
        out_specs=pl.BlockSpec(memory_space=pltpu.VMEM),
        scratch_shapes=[
            pltpu.VMEM((t, d), jnp.bfloat16),
            pltpu.VMEM((t, 1), jnp.int32),
            pltpu.VMEM((t, d), jnp.bfloat16),
            pltpu.VMEM((t, d), jnp.bfloat16),
            pltpu.SemaphoreType.DMA((3,)),
            pltpu.SemaphoreType.DMA((3,)),
        ],
        compiler_params=pltpu.CompilerParams(collective_id=0),
    )(x_bf, a2, w1_bf, w2_bf)


# device time: 84946 ns/iter; 1.1711x vs baseline; 1.1711x over previous
import jax
import jax.numpy as jnp
from jax import lax
from jax.experimental import pallas as pl
from jax.experimental.pallas import tpu as pltpu

N_CHUNKS = 4


def kernel(x, assign, W1, W2):
    t, d = x.shape
    n_loc, _, f = W1.shape
    tc = t // N_CHUNKS

    w1_bf = W1.astype(jnp.bfloat16)
    w2_bf = W2.astype(jnp.bfloat16)
    a2 = assign.reshape(t, 1)

    def body(x_ref, a_ref, w1_ref, w2_ref, out_ref,
             xs_buf, xr_buf, ar_buf, rs_buf, rr_buf, send_sems, recv_sems):
        my_x = lax.axis_index("x")
        my_y = lax.axis_index("y")
        my_z = lax.axis_index("z")
        peer = (my_x, 1 - my_y, my_z)

        barrier_sem = pltpu.get_barrier_semaphore()
        pl.semaphore_signal(barrier_sem, inc=1, device_id=peer,
                            device_id_type=pl.DeviceIdType.MESH)
        pl.semaphore_wait(barrier_sem, 1)

        xs_buf[...] = x_ref[...].astype(jnp.bfloat16)
        rdma_x = pltpu.make_async_remote_copy(
            src_ref=xs_buf, dst_ref=xr_buf,
            send_sem=send_sems.at[0], recv_sem=recv_sems.at[0],
            device_id=peer, device_id_type=pl.DeviceIdType.MESH)
        rdma_a = pltpu.make_async_remote_copy(
            src_ref=a_ref, dst_ref=ar_buf,
            send_sem=send_sems.at[1], recv_sem=recv_sems.at[1],
            device_id=peer, device_id_type=pl.DeviceIdType.MESH)
        rdma_x.start()
        rdma_a.start()

        e0 = 2 * my_y

        def ffn(xm, e):
            h = jnp.maximum(
                jnp.dot(xm, w1_ref[e], preferred_element_type=jnp.float32),
                0.0).astype(jnp.bfloat16)
            return jnp.dot(h, w2_ref[e], preferred_element_type=jnp.float32)

        def moe(xs, a):
            acc = ffn(jnp.where(a == e0, xs, 0), 0)
            for i in range(1, n_loc):
                acc = acc + ffn(jnp.where(a == e0 + i, xs, 0), i)
            return acc

        acc_local = moe(xs_buf[...], a_ref[...])

        rdma_x.wait()
        rdma_a.wait()

        rdma_r = []
        for c in range(N_CHUNKS):
            rows = pl.ds(c * tc, tc)
            acc_c = moe(xr_buf[rows, :], ar_buf[rows, :])
            rs_buf[rows, :] = acc_c.astype(jnp.bfloat16)
            r = pltpu.make_async_remote_copy(
                src_ref=rs_buf.at[rows, :], dst_ref=rr_buf.at[rows, :],
                send_sem=send_sems.at[2 + c], recv_sem=recv_sems.at[2 + c],
                device_id=peer, device_id_type=pl.DeviceIdType.MESH)
            r.start()
            rdma_r.append(r)

        for r in rdma_r:
            r.wait()
        out_ref[...] = acc_local + rr_buf[...].astype(jnp.float32)

    return pl.pallas_call(
        body,
        out_shape=jax.ShapeDtypeStruct((t, d), jnp.float32),
        in_specs=[pl.BlockSpec(memory_space=pltpu.VMEM)] * 4,
        out_specs=pl.BlockSpec(memory_space=pltpu.VMEM),
        scratch_shapes=[
            pltpu.VMEM((t, d), jnp.bfloat16),
            pltpu.VMEM((t, d), jnp.bfloat16),
            pltpu.VMEM((t, 1), jnp.int32),
            pltpu.VMEM((t, d), jnp.bfloat16),
            pltpu.VMEM((t, d), jnp.bfloat16),
            pltpu.SemaphoreType.DMA((2 + N_CHUNKS,)),
            pltpu.SemaphoreType.DMA((2 + N_CHUNKS,)),
        ],
        compiler_params=pltpu.CompilerParams(collective_id=0),
    )(x, a2, w1_bf, w2_bf)
